# baseline (speedup 1.0000x reference)
import functools

import jax
import jax.numpy as jnp
from jax import lax
from jax.experimental import pallas as pl
from jax.experimental.pallas import tpu as pltpu
from jax.experimental.pallas import tpu_sc as plsc

B = 2048
IN_DIM = 2048
N_GENES = 512
N_DONORS = 8
M_BLK = 128
N_BLOCKS = B // M_BLK + N_DONORS - 1
PAD_ROWS = N_BLOCKS * M_BLK
GRP = 16
GLANES = B // GRP


def _routing_body(labels_ref, ppos_ref, gid_ref):
    labels = labels_ref[...]
    lab128 = jnp.concatenate([labels] * N_DONORS, axis=0)
    r_iota = lax.broadcasted_iota(jnp.int32, (N_DONORS * GRP, GLANES), 0)
    d_of_row = r_iota // GRP
    oh = (lab128 == d_of_row).astype(jnp.bfloat16)

    j_a = lax.broadcasted_iota(jnp.int32, (GLANES, GLANES), 0)
    j_b = lax.broadcasted_iota(jnp.int32, (GLANES, GLANES), 1)
    upper_incl = (j_a <= j_b).astype(jnp.bfloat16)
    intra = lax.dot_general(oh, upper_incl, (((1,), (0,)), ((), ())),
                            preferred_element_type=jnp.float32)
    total = intra[:, GLANES - 1:GLANES]

    s_a = lax.broadcasted_iota(jnp.int32, (N_DONORS * GRP, N_DONORS * GRP), 0)
    s_b = lax.broadcasted_iota(jnp.int32, (N_DONORS * GRP, N_DONORS * GRP), 1)
    same_band = (s_a // GRP) == (s_b // GRP)
    strict = jnp.logical_and(same_band, s_b < s_a).astype(jnp.bfloat16)
    grpoff = lax.dot_general(strict, total.astype(jnp.bfloat16),
                             (((1,), (0,)), ((), ())),
                             preferred_element_type=jnp.float32)
    csum = intra + grpoff

    e_d = lax.broadcasted_iota(jnp.int32, (N_DONORS, N_DONORS * GRP), 0)
    e_r = lax.broadcasted_iota(jnp.int32, (N_DONORS, N_DONORS * GRP), 1)
    band_sel = (e_d == e_r // GRP).astype(jnp.bfloat16)
    counts = lax.dot_general(band_sel, total.astype(jnp.bfloat16),
                             (((1,), (0,)), ((), ())),
                             preferred_element_type=jnp.float32)

    nblk = jnp.floor((counts + (M_BLK - 1)) * (1.0 / M_BLK))
    t_a = lax.broadcasted_iota(jnp.int32, (N_DONORS, N_DONORS), 0)
    t_b = lax.broadcasted_iota(jnp.int32, (N_DONORS, N_DONORS), 1)
    s8 = (t_b < t_a).astype(jnp.bfloat16)
    pblk_off = lax.dot_general(s8, nblk.astype(jnp.bfloat16),
                               (((1,), (0,)), ((), ())),
                               preferred_element_type=jnp.float32)

    f_r = lax.broadcasted_iota(jnp.int32, (N_DONORS * GRP, N_DONORS), 0)
    f_d = lax.broadcasted_iota(jnp.int32, (N_DONORS * GRP, N_DONORS), 1)
    tile_sel = (f_r // GRP == f_d).astype(jnp.bfloat16)
    poff128 = lax.dot_general(tile_sel, pblk_off.astype(jnp.bfloat16),
                              (((1,), (0,)), ((), ())),
                              preferred_element_type=jnp.float32) * float(M_BLK)

    pre = oh.astype(jnp.float32) * (csum - 1.0 + poff128)
    ppos = jnp.sum(pre.reshape(N_DONORS, GRP, GLANES), axis=0)
    ppos_ref[...] = ppos.astype(jnp.int32)

    blk_iota = lax.broadcasted_iota(jnp.int32, (N_DONORS, 128), 1)
    cmp = (pblk_off.astype(jnp.int32) <= blk_iota).astype(jnp.int32)
    gid_ref[...] = jnp.sum(cmp, axis=0, keepdims=True) - 1


def _routing(labels):
    return pl.pallas_call(
        _routing_body,
        in_specs=[pl.BlockSpec((GRP, GLANES), lambda: (0, 0))],
        out_specs=[
            pl.BlockSpec((GRP, GLANES), lambda: (0, 0)),
            pl.BlockSpec((1, 128), lambda: (0, 0)),
        ],
        out_shape=[
            jax.ShapeDtypeStruct((GRP, GLANES), jnp.int32),
            jax.ShapeDtypeStruct((1, 128), jnp.int32),
        ],
    )(labels.reshape(GRP, GLANES))


def _sc_mesh():
    return plsc.VectorSubcoreMesh(core_axis_name="c", subcore_axis_name="s")


_NW = 32
_SCAT_CHUNK = 32


def _sc_scatter_x(x, ppos):
    per_w = B // _NW

    @functools.partial(
        pl.kernel,
        mesh=_sc_mesh(),
        out_type=jax.ShapeDtypeStruct((PAD_ROWS, IN_DIM), jnp.float32),
        scratch_types=[
            pltpu.VMEM((_SCAT_CHUNK,), jnp.int32),
            pltpu.VMEM((_SCAT_CHUNK, IN_DIM), jnp.float32),
        ],
    )
    def scatter_kernel(x_hbm, ppos_hbm, xp_hbm, idx_v, rows_v):
        wid = lax.axis_index("s") * 2 + lax.axis_index("c")
        base = wid * per_w

        @pl.loop(0, per_w, step=_SCAT_CHUNK)
        def _(c):
            pltpu.sync_copy(ppos_hbm.at[pl.ds(base + c, _SCAT_CHUNK)], idx_v)
            pltpu.sync_copy(x_hbm.at[pl.ds(base + c, _SCAT_CHUNK)], rows_v)
            pltpu.sync_copy(rows_v, xp_hbm.at[idx_v])

    return scatter_kernel(x, ppos)


def _sc_gather_out(y_padded, ppos):
    per_w = B // _NW

    @functools.partial(
        pl.kernel,
        mesh=_sc_mesh(),
        out_type=jax.ShapeDtypeStruct((B, N_GENES), jnp.float32),
        scratch_types=[
            pltpu.VMEM((per_w,), jnp.int32),
            pltpu.VMEM((per_w, N_GENES), jnp.float32),
            pltpu.SemaphoreType.DMA,
        ],
    )
    def gather_kernel(yp_hbm, ppos_hbm, out_hbm, idx_v, rows_v, sem):
        wid = lax.axis_index("s") * 2 + lax.axis_index("c")
        base = wid * per_w
        pltpu.sync_copy(ppos_hbm.at[pl.ds(base, per_w)], idx_v)
        pltpu.async_copy(yp_hbm.at[idx_v], rows_v, sem).wait()
        pltpu.sync_copy(rows_v, out_hbm.at[pl.ds(base, per_w)])

    return gather_kernel(y_padded, ppos)


def _matmul_body(gid_ref, x_ref, w_ref, b_ref, out_ref):
    out_ref[...] = lax.dot_general(
        x_ref[...], w_ref[0],
        dimension_numbers=(((1,), (1,)), ((), ())),
        preferred_element_type=jnp.float32,
    ) + b_ref[0]


def _grouped_matmul(x_padded, W, b, blk_gid):
    grid_spec = pltpu.PrefetchScalarGridSpec(
        num_scalar_prefetch=1,
        grid=(N_BLOCKS,),
        in_specs=[
            pl.BlockSpec((M_BLK, IN_DIM), lambda i, g: (i, 0)),
            pl.BlockSpec((1, N_GENES, IN_DIM), lambda i, g: (g[i], 0, 0)),
            pl.BlockSpec((1, 1, N_GENES), lambda i, g: (g[i], 0, 0)),
        ],
        out_specs=pl.BlockSpec((M_BLK, N_GENES), lambda i, g: (i, 0)),
    )
    return pl.pallas_call(
        _matmul_body,
        grid_spec=grid_spec,
        out_shape=jax.ShapeDtypeStruct((PAD_ROWS, N_GENES), jnp.float32),
    )(blk_gid, x_padded, W, b.reshape(N_DONORS, 1, N_GENES))


def kernel(x, donor_labels, W, b):
    ppos2d, gid2d = _routing(donor_labels)
    ppos = ppos2d.reshape(B)
    blk_gid = gid2d.reshape(128)
    x_padded = _sc_scatter_x(x, ppos)
    y_padded = _grouped_matmul(x_padded, W, b, blk_gid)
    return _sc_gather_out(y_padded, ppos)
